# baseline (speedup 1.0000x reference)
import functools
import jax
import jax.numpy as jnp
from jax import lax
from jax.experimental import pallas as pl
from jax.experimental.pallas import tpu as pltpu

N_CS = 10000; N_IN = 10000; E_CS = 160000; E_IN = 160000
D = 256; DE = 16; DG = 16; H = 256; B = 64


def _mm_kernel(x_ref, w_ref, o_ref):
    o_ref[...] = jnp.dot(x_ref[...], w_ref[...],
                         preferred_element_type=jnp.float32)


def _mm(x, w, bm=512):
    m, k = x.shape
    k2, n = w.shape
    grid = (pl.cdiv(m, bm),)
    return pl.pallas_call(
        _mm_kernel,
        grid=grid,
        in_specs=[pl.BlockSpec((bm, k), lambda i: (i, 0)),
                  pl.BlockSpec((k, n), lambda i: (0, 0))],
        out_specs=pl.BlockSpec((bm, n), lambda i: (i, 0)),
        out_shape=jax.ShapeDtypeStruct((m, n), jnp.float32),
    )(x, w)


def _seg_mean(vals, ids, num):
    sm = jax.ops.segment_sum(vals, ids, num_segments=num)
    cn = jax.ops.segment_sum(jnp.ones((vals.shape[0],), vals.dtype), ids,
                             num_segments=num)
    return sm / jnp.maximum(cn, 1.0)[:, None]


def kernel(cs_x, in_x, cs_edge_index, in_edge_sources, in_edge_targets,
           cs_edge_attr, in_edge_attr, global_attr, cs_node_batch,
           in_node_batch, W_gat, a_att, W_gat_out, Wp1, Wp2, We1, We2,
           Wn1, Wn2, Wg1, Wg2, Wr, Wm1, bm1, Wm2, bm2):
    h = _mm(in_x, W_gat)
    a1 = a_att[:H, 0]; a2 = a_att[H:2 * H, 0]; a3 = a_att[2 * H:, 0]
    s1 = h @ a1
    s2 = h @ a2
    e3 = in_edge_attr @ a3
    logits = s1[in_edge_sources] + s2[in_edge_targets] + e3
    logits = jnp.where(logits >= 0, logits, 0.2 * logits)
    ex = jnp.exp(logits)
    denom = jax.ops.segment_sum(ex, in_edge_targets, num_segments=N_IN)
    num = jax.ops.segment_sum(h[in_edge_sources] * ex[:, None],
                              in_edge_targets, num_segments=N_IN)
    agg = num / (denom[:, None] + 1e-16)
    node_in = jax.nn.relu(_mm(agg, W_gat_out))
    pooled_in = _seg_mean(node_in, in_node_batch, B)
    in_out = jax.nn.relu(jnp.concatenate([pooled_in, global_attr], 1) @ Wp1)
    in_out = jax.nn.relu(in_out @ Wp2)
    src = cs_edge_index[0]; dst = cs_edge_index[1]
    eb = cs_node_batch[src]
    p_src = _mm(cs_x, We1[:D])
    p_dst = _mm(cs_x, We1[D:2 * D])
    p_e = _mm(cs_edge_attr, We1[2 * D:2 * D + DE])
    p_g = global_attr @ We1[2 * D + DE:]
    e_pre = jax.nn.relu(p_src[src] + p_dst[dst] + p_e + p_g[eb])
    e_h = jax.nn.relu(_mm(e_pre, We2))
    e2n = _seg_mean(e_h, dst, N_CS)
    gpn = global_attr[cs_node_batch]
    n_pre = jax.nn.relu(_mm(cs_x, Wn1[:D]) + _mm(e2n, Wn1[D:2 * D])
                        + gpn @ Wn1[2 * D:])
    n_h = jax.nn.relu(_mm(n_pre, Wn2))
    node_mean = _seg_mean(n_h, cs_node_batch, B)
    edge_mean = _seg_mean(e_h, eb, B)
    gcat = jnp.concatenate([node_mean, edge_mean, global_attr], 1)
    g_h = jax.nn.relu(jax.nn.relu(gcat @ Wg1) @ Wg2)
    cs_out = jax.nn.relu(jnp.concatenate([node_mean, edge_mean, g_h], 1) @ Wr)
    merged = jnp.concatenate([in_out, cs_out], 1)
    hm = jax.nn.relu(merged @ Wm1 + bm1)
    final = hm @ Wm2 + bm2
    return final.reshape(-1)
